# Initial kernel scaffold; baseline (speedup 1.0000x reference)
import jax
import jax.numpy as jnp
from jax.experimental import pallas as pl


def kernel(x, tok_table, pos_table):
    raise NotImplementedError("write your pallas kernel here")


# speedup vs baseline: 1.0231x; 1.0231x over previous
import functools

import jax
import jax.numpy as jnp
from jax import lax
from jax.experimental import pallas as pl
from jax.experimental.pallas import tpu as pltpu
from jax.experimental.pallas import tpu_sc as plsc

_BATCH, _SEQ, _EMBED = 4, 2048, 768
_ROWS = _BATCH * _SEQ
_NW = 32
_RPW = _ROWS // _NW
_CH = 64
_NCH = _RPW // _CH
_LANES = 16
_VPR = _EMBED // _LANES


def _emb_body(x_hbm, tok_hbm, pos_hbm, out_hbm, idx_v, tbuf, obuf, sem):
    c = lax.axis_index("c")
    s = lax.axis_index("s")
    wid = s * 2 + c
    base = wid * _RPW
    pltpu.sync_copy(x_hbm.at[pl.ds(base, _RPW)], idx_v)
    for ci in range(_NCH):
        rb = base + ci * _CH
        pb = lax.rem(rb, _SEQ)
        gather = pltpu.async_copy(
            tok_hbm.at[idx_v.at[pl.ds(ci * _CH, _CH)]], tbuf, sem
        )
        pltpu.sync_copy(pos_hbm.at[pl.ds(pb, _CH)], obuf)
        gather.wait()

        def row_add(i, carry):
            for j in range(_VPR):
                sl = pl.ds(j * _LANES, _LANES)
                plsc.addupdate(obuf.at[i, sl], tbuf[i, sl])
            return carry

        lax.fori_loop(0, _CH, row_add, 0)
        pltpu.sync_copy(obuf, out_hbm.at[pl.ds(rb, _CH)])


@jax.jit
def kernel(x, tok_table, pos_table):
    xf = x.reshape(_ROWS)
    mesh = plsc.VectorSubcoreMesh(core_axis_name="c", subcore_axis_name="s")
    fn = pl.kernel(
        _emb_body,
        out_type=jax.ShapeDtypeStruct((_ROWS, _EMBED), jnp.float32),
        mesh=mesh,
        scratch_types=[
            pltpu.VMEM((_RPW,), jnp.int32),
            pltpu.VMEM((_CH, _EMBED), jnp.float32),
            pltpu.VMEM((_CH, _EMBED), jnp.float32),
            pltpu.SemaphoreType.DMA,
        ],
    )
    out = fn(xf, tok_table, pos_table)
    return out.reshape(_BATCH, _SEQ, _EMBED)


# speedup vs baseline: 1.2035x; 1.1764x over previous
import functools

import jax
import jax.numpy as jnp
from jax import lax
from jax.experimental import pallas as pl
from jax.experimental.pallas import tpu as pltpu
from jax.experimental.pallas import tpu_sc as plsc

_BATCH, _SEQ, _EMBED = 4, 2048, 768
_ROWS = _BATCH * _SEQ
_NW = 32
_RPW = _ROWS // _NW
_CH = 32
_NCH = _RPW // _CH
_LANES = 16
_VPR = _EMBED // _LANES


def _emb_body(x_hbm, tok_hbm, pos_hbm, out_hbm, idx_v,
              tbuf0, tbuf1, obuf0, obuf1,
              gsem0, gsem1, psem0, psem1, wsem0, wsem1):
    tbufs = (tbuf0, tbuf1)
    obufs = (obuf0, obuf1)
    gsems = (gsem0, gsem1)
    psems = (psem0, psem1)
    wsems = (wsem0, wsem1)

    c = lax.axis_index("c")
    s = lax.axis_index("s")
    wid = s * 2 + c
    base = wid * _RPW
    pltpu.sync_copy(x_hbm.at[pl.ds(base, _RPW)], idx_v)

    def start_gather(ci, b):
        return pltpu.async_copy(
            tok_hbm.at[idx_v.at[pl.ds(ci * _CH, _CH)]], tbufs[b], gsems[b]
        )

    def start_pos(ci, b):
        pb = lax.rem(base + ci * _CH, _SEQ)
        return pltpu.async_copy(pos_hbm.at[pl.ds(pb, _CH)], obufs[b], psems[b])

    ghandles = {0: start_gather(0, 0), 1: start_gather(1, 1)}
    phandles = {0: start_pos(0, 0), 1: start_pos(1, 1)}
    whandles = {}

    for ci in range(_NCH):
        b = ci % 2
        ghandles.pop(ci).wait()
        phandles.pop(ci).wait()

        def row_add(i, carry):
            for j in range(_VPR):
                sl = pl.ds(j * _LANES, _LANES)
                plsc.addupdate(obufs[b].at[i, sl], tbufs[b][i, sl])
            return carry

        lax.fori_loop(0, _CH, row_add, 0)

        rb = base + ci * _CH
        whandles[ci] = pltpu.async_copy(
            obufs[b], out_hbm.at[pl.ds(rb, _CH)], wsems[b]
        )

        if ci + 2 < _NCH:
            ghandles[ci + 2] = start_gather(ci + 2, b)
            whandles.pop(ci).wait()
            phandles[ci + 2] = start_pos(ci + 2, b)

    whandles.pop(_NCH - 2).wait()
    whandles.pop(_NCH - 1).wait()


@jax.jit
def kernel(x, tok_table, pos_table):
    xf = x.reshape(_ROWS)
    mesh = plsc.VectorSubcoreMesh(core_axis_name="c", subcore_axis_name="s")
    fn = pl.kernel(
        _emb_body,
        out_type=jax.ShapeDtypeStruct((_ROWS, _EMBED), jnp.float32),
        mesh=mesh,
        scratch_types=[
            pltpu.VMEM((_RPW,), jnp.int32),
            pltpu.VMEM((_CH, _EMBED), jnp.float32),
            pltpu.VMEM((_CH, _EMBED), jnp.float32),
            pltpu.VMEM((_CH, _EMBED), jnp.float32),
            pltpu.VMEM((_CH, _EMBED), jnp.float32),
            pltpu.SemaphoreType.DMA,
            pltpu.SemaphoreType.DMA,
            pltpu.SemaphoreType.DMA,
            pltpu.SemaphoreType.DMA,
            pltpu.SemaphoreType.DMA,
            pltpu.SemaphoreType.DMA,
        ],
    )
    out = fn(xf, tok_table, pos_table)
    return out.reshape(_BATCH, _SEQ, _EMBED)


# speedup vs baseline: 1.3188x; 1.0957x over previous
import functools

import jax
import jax.numpy as jnp
from jax import lax
from jax.experimental import pallas as pl
from jax.experimental.pallas import tpu as pltpu
from jax.experimental.pallas import tpu_sc as plsc

_BATCH, _SEQ, _EMBED = 4, 2048, 768
_NW = 32
_PPW = _SEQ // _NW
_CP = 8
_NCH = _PPW // _CP
_NTB = 3
_NPB = 2
_LANES = 16
_VPR = _EMBED // _LANES
_GRP = 16


def _emb_body(x_hbm, tok_hbm, pos_hbm, out_hbm, idx_v,
              tbuf0, tbuf1, tbuf2, pbuf0, pbuf1,
              gsem0, gsem1, gsem2, psem0, psem1, wsem0, wsem1, wsem2):
    tbufs = (tbuf0, tbuf1, tbuf2)
    pbufs = (pbuf0, pbuf1)
    gsems = (gsem0, gsem1, gsem2)
    psems = (psem0, psem1)
    wsems = (wsem0, wsem1, wsem2)

    c = lax.axis_index("c")
    s = lax.axis_index("s")
    wid = s * 2 + c
    p0 = wid * _PPW

    for b in range(_BATCH):
        pltpu.sync_copy(
            x_hbm.at[pl.ds(b * _SEQ + p0, _PPW)],
            idx_v.at[pl.ds(b * _PPW, _PPW)],
        )

    def start_gathers(ci, rb):
        hs = []
        for b in range(_BATCH):
            hs.append(pltpu.async_copy(
                tok_hbm.at[idx_v.at[pl.ds(b * _PPW + ci * _CP, _CP)]],
                tbufs[rb].at[b],
                gsems[rb],
            ))
        return hs

    def start_pos(ci, rb):
        return pltpu.async_copy(
            pos_hbm.at[pl.ds(p0 + ci * _CP, _CP)], pbufs[rb], psems[rb]
        )

    ghandles = {ci: start_gathers(ci, ci % _NTB) for ci in range(_NTB)}
    phandles = {ci: start_pos(ci, ci % _NPB) for ci in range(_NPB)}
    whandles = {}

    for ci in range(_NCH):
        tb = ci % _NTB
        pb = ci % _NPB
        for h in ghandles.pop(ci):
            h.wait()
        phandles.pop(ci).wait()

        def pos_add(i, carry, tb=tb, pb=pb):
            for g in range(_VPR // _GRP):
                pvecs = [
                    pbufs[pb][i, pl.ds((g * _GRP + k) * _LANES, _LANES)]
                    for k in range(_GRP)
                ]
                for b in range(_BATCH):
                    for k in range(_GRP):
                        sl = pl.ds((g * _GRP + k) * _LANES, _LANES)
                        tbufs[tb][b, i, sl] = tbufs[tb][b, i, sl] + pvecs[k]
            return carry

        lax.fori_loop(0, _CP, pos_add, 0)

        whandles[ci] = [
            pltpu.async_copy(
                tbufs[tb].at[b],
                out_hbm.at[pl.ds(b * _SEQ + p0 + ci * _CP, _CP)],
                wsems[tb],
            )
            for b in range(_BATCH)
        ]

        if ci + _NPB < _NCH:
            phandles[ci + _NPB] = start_pos(ci + _NPB, pb)
        if ci + _NTB < _NCH:
            for h in whandles.pop(ci):
                h.wait()
            ghandles[ci + _NTB] = start_gathers(ci + _NTB, tb)

    for ci in sorted(whandles):
        for h in whandles.pop(ci):
            h.wait()


@jax.jit
def kernel(x, tok_table, pos_table):
    xf = x.reshape(_BATCH * _SEQ)
    mesh = plsc.VectorSubcoreMesh(core_axis_name="c", subcore_axis_name="s")
    fn = pl.kernel(
        _emb_body,
        out_type=jax.ShapeDtypeStruct((_BATCH * _SEQ, _EMBED), jnp.float32),
        mesh=mesh,
        scratch_types=[
            pltpu.VMEM((_BATCH * _PPW,), jnp.int32),
            pltpu.VMEM((_BATCH, _CP, _EMBED), jnp.float32),
            pltpu.VMEM((_BATCH, _CP, _EMBED), jnp.float32),
            pltpu.VMEM((_BATCH, _CP, _EMBED), jnp.float32),
            pltpu.VMEM((_CP, _EMBED), jnp.float32),
            pltpu.VMEM((_CP, _EMBED), jnp.float32),
            pltpu.SemaphoreType.DMA,
            pltpu.SemaphoreType.DMA,
            pltpu.SemaphoreType.DMA,
            pltpu.SemaphoreType.DMA,
            pltpu.SemaphoreType.DMA,
            pltpu.SemaphoreType.DMA,
            pltpu.SemaphoreType.DMA,
            pltpu.SemaphoreType.DMA,
        ],
    )
    out = fn(xf, tok_table, pos_table)
    return out.reshape(_BATCH, _SEQ, _EMBED)
